# baseline (speedup 1.0000x reference)
import functools

import jax
import jax.numpy as jnp
from jax import lax
from jax.experimental import pallas as pl
from jax.experimental.pallas import tpu as pltpu
from jax.experimental.pallas import tpu_sc as plsc

_WB = 2048


def _tp_body(t_ref, o_ref):
    o_ref[:, 0:64] = t_ref[...].T


def _row_major_table(table_t):
    V = table_t.shape[1]
    grid = pl.cdiv(V, _WB)
    return pl.pallas_call(
        _tp_body,
        grid=(grid,),
        in_specs=[pl.BlockSpec((64, _WB), lambda g: (0, g))],
        out_specs=pl.BlockSpec((_WB, 128), lambda g: (g, 0)),
        out_shape=jax.ShapeDtypeStruct((V, 128), jnp.float32),
    )(table_t)


def _make_sc_gather(B, L, D, V):
    info = plsc.get_sparse_core_info()
    NC, NS = info.num_cores, info.num_subcores
    NW = NC * NS
    b_per_w = B // NW
    assert b_per_w == 128 and D == 64
    mesh = plsc.VectorSubcoreMesh(core_axis_name="c", subcore_axis_name="s")

    @functools.partial(
        pl.kernel,
        mesh=mesh,
        out_type=jax.ShapeDtypeStruct((L, D, B), jnp.float32),
        scratch_types=(
            [pltpu.VMEM((L, 128), jnp.int32)]
            + [pltpu.VMEM((128, 128), jnp.float32) for _ in range(4)]
            + [pltpu.VMEM((D, 128), jnp.float32) for _ in range(4)]
            + [pltpu.SemaphoreType.DMA for _ in range(8)]
        ),
        compiler_params=pltpu.CompilerParams(
            use_tc_tiling_on_sc=True, needs_layout_passes=False
        ),
    )
    def sc_gather(
        xt_hbm, tab_hbm, out_hbm, idxv, g0, g1, g2, g3, o0, o1, o2, o3, *sems
    ):
        pairs = (g0, g1, g2, g3)
        outblk = (o0, o1, o2, o3)
        gsem = sems[:4]
        osem = sems[4:]
        wid = lax.axis_index("s") * NC + lax.axis_index("c")
        b0 = wid * b_per_w
        pltpu.sync_copy(xt_hbm.at[:, pl.ds(b0, b_per_w)], idxv)

        def start_gather(l, b):
            pltpu.async_copy(tab_hbm.at[idxv.at[l]], pairs[b], gsem[b])

        def wait_gather(l, b):
            pltpu.make_async_copy(
                tab_hbm.at[idxv.at[l]], pairs[b], gsem[b]
            ).wait()

        def start_out(l, b):
            pltpu.async_copy(
                outblk[b], out_hbm.at[l, :, pl.ds(b0, b_per_w)], osem[b]
            )

        def wait_out(l, b):
            pltpu.make_async_copy(
                outblk[b], out_hbm.at[l, :, pl.ds(b0, b_per_w)], osem[b]
            ).wait()

        lanes = lax.iota(jnp.int32, 16)
        skew = [jnp.bitwise_and(lanes + k, 15) for k in range(16)]

        def transpose_block(src, dst):
            def g_body(g, c):
                rows = lanes + g * 16
                for d0 in range(0, D, 16):
                    for k0 in range(0, 16, 8):
                        cvs = []
                        for k in range(k0, k0 + 8):
                            cols = skew[k] + d0
                            cvs.append(
                                (cols, plsc.load_gather(src, [rows, cols]))
                            )
                        for cols, v in cvs:
                            plsc.store_scatter(dst, [cols, rows], v)
                return c

            lax.fori_loop(0, 8, g_body, 0)

        for b in range(3):
            start_gather(b, b)

        def quad_body(p, carry):
            for b in range(4):
                l = p * 4 + b
                wait_gather(l, b)

                @pl.when(l >= 4)
                def _():
                    wait_out(l - 4, b)

                transpose_block(pairs[b], outblk[b])
                start_out(l, b)

                @pl.when(l + 3 < L)
                def _():
                    start_gather(l + 3, (b + 3) % 4)

            return carry

        lax.fori_loop(0, L // 4, quad_body, 0)
        for t in range(L - 4, L):
            wait_out(t, t % 4)

    return sc_gather


def kernel(x, table):
    B, L = x.shape
    V, D = table.shape
    table_rm = _row_major_table(table.T)
    xt = x.T.astype(jnp.int32)
    out2 = _make_sc_gather(B, L, D, V)(xt, table_rm)
    return jnp.transpose(out2, (2, 0, 1))


# speedup vs baseline: 1.2321x; 1.2321x over previous
import functools

import jax
import jax.numpy as jnp
from jax import lax
from jax.experimental import pallas as pl
from jax.experimental.pallas import tpu as pltpu
from jax.experimental.pallas import tpu_sc as plsc

_WB = 4096


def _tp_body(t_ref, o_ref):
    o_ref[:, 0:64] = t_ref[...].T


def _row_major_table(table_t):
    V = table_t.shape[1]
    grid = pl.cdiv(V, _WB)
    return pl.pallas_call(
        _tp_body,
        grid=(grid,),
        in_specs=[pl.BlockSpec((64, _WB), lambda g: (0, g))],
        out_specs=pl.BlockSpec((_WB, 128), lambda g: (g, 0)),
        out_shape=jax.ShapeDtypeStruct((V, 128), jnp.float32),
    )(table_t)


def _make_sc_gather(B, L, D, V):
    info = plsc.get_sparse_core_info()
    NC, NS = info.num_cores, info.num_subcores
    NW = NC * NS
    b_per_w = B // NW
    assert b_per_w == 128 and D == 64
    mesh = plsc.VectorSubcoreMesh(core_axis_name="c", subcore_axis_name="s")

    @functools.partial(
        pl.kernel,
        mesh=mesh,
        out_type=jax.ShapeDtypeStruct((L, D, B), jnp.float32),
        scratch_types=(
            [pltpu.VMEM((L, 128), jnp.int32)]
            + [pltpu.VMEM((128, 128), jnp.float32) for _ in range(3)]
            + [pltpu.VMEM((D, 128), jnp.float32) for _ in range(3)]
            + [pltpu.SemaphoreType.DMA for _ in range(6)]
        ),
        compiler_params=pltpu.CompilerParams(
            use_tc_tiling_on_sc=True, needs_layout_passes=False
        ),
    )
    def sc_gather(
        xt_hbm, tab_hbm, out_hbm, idxv, g0, g1, g2, o0, o1, o2, *sems
    ):
        pairs = (g0, g1, g2)
        outblk = (o0, o1, o2)
        gsem = sems[:3]
        osem = sems[3:]
        wid = lax.axis_index("s") * NC + lax.axis_index("c")
        b0 = wid * b_per_w
        pltpu.sync_copy(xt_hbm.at[:, pl.ds(b0, b_per_w)], idxv)

        def start_gather(l, b):
            pltpu.async_copy(tab_hbm.at[idxv.at[l]], pairs[b], gsem[b])

        def wait_gather(l, b):
            pltpu.make_async_copy(
                tab_hbm.at[idxv.at[l]], pairs[b], gsem[b]
            ).wait()

        def start_out(l, b):
            pltpu.async_copy(
                outblk[b], out_hbm.at[l, :, pl.ds(b0, b_per_w)], osem[b]
            )

        def wait_out(l, b):
            pltpu.make_async_copy(
                outblk[b], out_hbm.at[l, :, pl.ds(b0, b_per_w)], osem[b]
            ).wait()

        lanes = lax.iota(jnp.int32, 16)
        skew = [jnp.bitwise_and(lanes + k, 15) for k in range(16)]

        def transpose_block(src, dst):
            def g_body(g, c):
                rows = lanes + g * 16
                for d0 in range(0, D, 16):
                    for k0 in range(0, 16, 8):
                        cvs = []
                        for k in range(k0, k0 + 8):
                            cols = skew[k] + d0
                            cvs.append(
                                (cols, plsc.load_gather(src, [rows, cols]))
                            )
                        for cols, v in cvs:
                            plsc.store_scatter(dst, [cols, rows], v)
                return c

            lax.fori_loop(0, 8, g_body, 0)

        start_gather(0, 0)
        start_gather(1, 1)
        start_gather(2, 2)

        def trip_body(p, carry):
            for b in range(3):
                l = p * 3 + b
                wait_gather(l, b)

                @pl.when(l >= 3)
                def _():
                    wait_out(l - 3, b)

                transpose_block(pairs[b], outblk[b])
                start_out(l, b)

                @pl.when(l + 3 < L)
                def _():
                    start_gather(l + 3, b)

            return carry

        lax.fori_loop(0, L // 3, trip_body, 0)

        for t, b in ((L - 2, (L - 2) % 3), (L - 1, (L - 1) % 3)):
            wait_gather(t, b)
            wait_out(t - 3, b)
            transpose_block(pairs[b], outblk[b])
            start_out(t, b)
        for t in range(L - 3, L):
            wait_out(t, t % 3)

    return sc_gather


def kernel(x, table):
    B, L = x.shape
    V, D = table.shape
    table_rm = _row_major_table(table.T)
    xt = x.T.astype(jnp.int32)
    out2 = _make_sc_gather(B, L, D, V)(xt, table_rm)
    return jnp.transpose(out2, (2, 0, 1))


# speedup vs baseline: 1.4070x; 1.1420x over previous
import functools

import jax
import jax.numpy as jnp
from jax import lax
from jax.experimental import pallas as pl
from jax.experimental.pallas import tpu as pltpu
from jax.experimental.pallas import tpu_sc as plsc

_WB = 8192


def _tp_body(t_ref, o_ref):
    o_ref[:, 0:64] = t_ref[...].T


def _row_major_table(table_t):
    V = table_t.shape[1]
    grid = pl.cdiv(V, _WB)
    return pl.pallas_call(
        _tp_body,
        grid=(grid,),
        in_specs=[pl.BlockSpec((64, _WB), lambda g: (0, g))],
        out_specs=pl.BlockSpec((_WB, 128), lambda g: (g, 0)),
        out_shape=jax.ShapeDtypeStruct((V, 128), jnp.float32),
    )(table_t)


def _make_sc_gather(B, L, D, V):
    info = plsc.get_sparse_core_info()
    NC, NS = info.num_cores, info.num_subcores
    NW = NC * NS
    b_per_w = B // NW
    assert b_per_w == 128 and D == 64
    mesh = plsc.VectorSubcoreMesh(core_axis_name="c", subcore_axis_name="s")

    @functools.partial(
        pl.kernel,
        mesh=mesh,
        out_type=jax.ShapeDtypeStruct((L, D, B), jnp.float32),
        scratch_types=(
            [pltpu.VMEM((L, 128), jnp.int32)]
            + [pltpu.VMEM((128, 128), jnp.float32) for _ in range(3)]
            + [pltpu.VMEM((D, 128), jnp.float32) for _ in range(3)]
            + [pltpu.SemaphoreType.DMA for _ in range(6)]
        ),
        compiler_params=pltpu.CompilerParams(
            use_tc_tiling_on_sc=True, needs_layout_passes=False
        ),
    )
    def sc_gather(
        xt_hbm, tab_hbm, out_hbm, idxv, g0, g1, g2, o0, o1, o2, *sems
    ):
        pairs = (g0, g1, g2)
        outblk = (o0, o1, o2)
        gsem = sems[:3]
        osem = sems[3:]
        wid = lax.axis_index("s") * NC + lax.axis_index("c")
        b0 = wid * b_per_w
        pltpu.sync_copy(xt_hbm.at[:, pl.ds(b0, b_per_w)], idxv)

        def start_gather(l, b):
            pltpu.async_copy(tab_hbm.at[idxv.at[l]], pairs[b], gsem[b])

        def wait_gather(l, b):
            pltpu.make_async_copy(
                tab_hbm.at[idxv.at[l]], pairs[b], gsem[b]
            ).wait()

        def start_out(l, b):
            pltpu.async_copy(
                outblk[b], out_hbm.at[l, :, pl.ds(b0, b_per_w)], osem[b]
            )

        def wait_out(l, b):
            pltpu.make_async_copy(
                outblk[b], out_hbm.at[l, :, pl.ds(b0, b_per_w)], osem[b]
            ).wait()

        lanes = lax.iota(jnp.int32, 16)
        skew = [jnp.bitwise_and(lanes + k, 15) for k in range(16)]

        def transpose_block(src, dst):
            def g_body(g, c):
                rows = lanes + g * 16
                for d0 in range(0, D, 16):
                    for k0 in range(0, 16, 8):
                        cvs = []
                        for k in range(k0, k0 + 8):
                            cols = skew[k] + d0
                            cvs.append(
                                (cols, plsc.load_gather(src, [rows, cols]))
                            )
                        for cols, v in cvs:
                            plsc.store_scatter(dst, [cols, rows], v)
                return c

            lax.fori_loop(0, 8, g_body, 0)

        start_gather(0, 0)
        start_gather(1, 1)
        start_gather(2, 2)

        def trip_body(p, carry):
            for b in range(3):
                l = p * 3 + b
                wait_gather(l, b)

                @pl.when(l >= 3)
                def _():
                    wait_out(l - 3, b)

                transpose_block(pairs[b], outblk[b])
                start_out(l, b)

                @pl.when(l + 3 < L)
                def _():
                    start_gather(l + 3, b)

            return carry

        lax.fori_loop(0, L // 3, trip_body, 0)

        for t, b in ((L - 2, (L - 2) % 3), (L - 1, (L - 1) % 3)):
            wait_gather(t, b)
            wait_out(t - 3, b)
            transpose_block(pairs[b], outblk[b])
            start_out(t, b)
        for t in range(L - 3, L):
            wait_out(t, t % 3)

    return sc_gather


def kernel(x, table):
    B, L = x.shape
    V, D = table.shape
    table_rm = _row_major_table(table.T)
    xt = x.T.astype(jnp.int32)
    out2 = _make_sc_gather(B, L, D, V)(xt, table_rm)
    return jnp.transpose(out2, (2, 0, 1))


# speedup vs baseline: 1.4628x; 1.0397x over previous
import functools

import jax
import jax.numpy as jnp
from jax import lax
from jax.experimental import pallas as pl
from jax.experimental.pallas import tpu as pltpu
from jax.experimental.pallas import tpu_sc as plsc

_WB = 16384


def _tp_body(t_ref, o_ref):
    o_ref[:, 0:64] = t_ref[...].T


def _row_major_table(table_t):
    V = table_t.shape[1]
    grid = pl.cdiv(V, _WB)
    return pl.pallas_call(
        _tp_body,
        grid=(grid,),
        in_specs=[pl.BlockSpec((64, _WB), lambda g: (0, g))],
        out_specs=pl.BlockSpec((_WB, 128), lambda g: (g, 0)),
        out_shape=jax.ShapeDtypeStruct((V, 128), jnp.float32),
    )(table_t)


def _make_sc_gather(B, L, D, V):
    info = plsc.get_sparse_core_info()
    NC, NS = info.num_cores, info.num_subcores
    NW = NC * NS
    b_per_w = B // NW
    assert b_per_w == 128 and D == 64
    mesh = plsc.VectorSubcoreMesh(core_axis_name="c", subcore_axis_name="s")

    @functools.partial(
        pl.kernel,
        mesh=mesh,
        out_type=jax.ShapeDtypeStruct((L, D, B), jnp.float32),
        scratch_types=(
            [pltpu.VMEM((L, 128), jnp.int32)]
            + [pltpu.VMEM((128, 128), jnp.float32) for _ in range(3)]
            + [pltpu.VMEM((D, 128), jnp.float32) for _ in range(3)]
            + [pltpu.SemaphoreType.DMA for _ in range(6)]
        ),
        compiler_params=pltpu.CompilerParams(
            use_tc_tiling_on_sc=True, needs_layout_passes=False
        ),
    )
    def sc_gather(
        xt_hbm, tab_hbm, out_hbm, idxv, g0, g1, g2, o0, o1, o2, *sems
    ):
        pairs = (g0, g1, g2)
        outblk = (o0, o1, o2)
        gsem = sems[:3]
        osem = sems[3:]
        wid = lax.axis_index("s") * NC + lax.axis_index("c")
        b0 = wid * b_per_w
        pltpu.sync_copy(xt_hbm.at[:, pl.ds(b0, b_per_w)], idxv)

        def start_gather(l, b):
            pltpu.async_copy(tab_hbm.at[idxv.at[l]], pairs[b], gsem[b])

        def wait_gather(l, b):
            pltpu.make_async_copy(
                tab_hbm.at[idxv.at[l]], pairs[b], gsem[b]
            ).wait()

        def start_out(l, b):
            pltpu.async_copy(
                outblk[b], out_hbm.at[l, :, pl.ds(b0, b_per_w)], osem[b]
            )

        def wait_out(l, b):
            pltpu.make_async_copy(
                outblk[b], out_hbm.at[l, :, pl.ds(b0, b_per_w)], osem[b]
            ).wait()

        lanes = lax.iota(jnp.int32, 16)
        skew = [jnp.bitwise_and(lanes + k, 15) for k in range(16)]

        def transpose_block(src, dst):
            def g_body(g, c):
                rows = lanes + g * 16
                for d0 in range(0, D, 16):
                    for k0 in range(0, 16, 8):
                        cvs = []
                        for k in range(k0, k0 + 8):
                            cols = skew[k] + d0
                            cvs.append(
                                (cols, plsc.load_gather(src, [rows, cols]))
                            )
                        for cols, v in cvs:
                            plsc.store_scatter(dst, [cols, rows], v)
                return c

            lax.fori_loop(0, 8, g_body, 0)

        start_gather(0, 0)
        start_gather(1, 1)
        start_gather(2, 2)

        def trip_body(p, carry):
            for b in range(3):
                l = p * 3 + b
                wait_gather(l, b)

                @pl.when(l >= 3)
                def _():
                    wait_out(l - 3, b)

                transpose_block(pairs[b], outblk[b])
                start_out(l, b)

                @pl.when(l + 3 < L)
                def _():
                    start_gather(l + 3, b)

            return carry

        lax.fori_loop(0, L // 3, trip_body, 0)

        for t, b in ((L - 2, (L - 2) % 3), (L - 1, (L - 1) % 3)):
            wait_gather(t, b)
            wait_out(t - 3, b)
            transpose_block(pairs[b], outblk[b])
            start_out(t, b)
        for t in range(L - 3, L):
            wait_out(t, t % 3)

    return sc_gather


def kernel(x, table):
    B, L = x.shape
    V, D = table.shape
    table_rm = _row_major_table(table.T)
    xt = x.T.astype(jnp.int32)
    out2 = _make_sc_gather(B, L, D, V)(xt, table_rm)
    return jnp.transpose(out2, (2, 0, 1))


# speedup vs baseline: 1.4817x; 1.0129x over previous
import functools

import jax
import jax.numpy as jnp
from jax import lax
from jax.experimental import pallas as pl
from jax.experimental.pallas import tpu as pltpu
from jax.experimental.pallas import tpu_sc as plsc

_WB = 32768


def _tp_body(t_ref, o_ref):
    o_ref[:, 0:64] = t_ref[...].T


def _row_major_table(table_t):
    V = table_t.shape[1]
    grid = pl.cdiv(V, _WB)
    return pl.pallas_call(
        _tp_body,
        grid=(grid,),
        in_specs=[pl.BlockSpec((64, _WB), lambda g: (0, g))],
        out_specs=pl.BlockSpec((_WB, 128), lambda g: (g, 0)),
        out_shape=jax.ShapeDtypeStruct((V, 128), jnp.float32),
    )(table_t)


def _make_sc_gather(B, L, D, V):
    info = plsc.get_sparse_core_info()
    NC, NS = info.num_cores, info.num_subcores
    NW = NC * NS
    b_per_w = B // NW
    assert b_per_w == 128 and D == 64
    mesh = plsc.VectorSubcoreMesh(core_axis_name="c", subcore_axis_name="s")

    @functools.partial(
        pl.kernel,
        mesh=mesh,
        out_type=jax.ShapeDtypeStruct((L, D, B), jnp.float32),
        scratch_types=(
            [pltpu.VMEM((L, 128), jnp.int32)]
            + [pltpu.VMEM((128, 128), jnp.float32) for _ in range(3)]
            + [pltpu.VMEM((D, 128), jnp.float32) for _ in range(3)]
            + [pltpu.SemaphoreType.DMA for _ in range(6)]
        ),
        compiler_params=pltpu.CompilerParams(
            use_tc_tiling_on_sc=True, needs_layout_passes=False
        ),
    )
    def sc_gather(
        xt_hbm, tab_hbm, out_hbm, idxv, g0, g1, g2, o0, o1, o2, *sems
    ):
        pairs = (g0, g1, g2)
        outblk = (o0, o1, o2)
        gsem = sems[:3]
        osem = sems[3:]
        wid = lax.axis_index("s") * NC + lax.axis_index("c")
        b0 = wid * b_per_w
        pltpu.sync_copy(xt_hbm.at[:, pl.ds(b0, b_per_w)], idxv)

        def start_gather(l, b):
            pltpu.async_copy(tab_hbm.at[idxv.at[l]], pairs[b], gsem[b])

        def wait_gather(l, b):
            pltpu.make_async_copy(
                tab_hbm.at[idxv.at[l]], pairs[b], gsem[b]
            ).wait()

        def start_out(l, b):
            pltpu.async_copy(
                outblk[b], out_hbm.at[l, :, pl.ds(b0, b_per_w)], osem[b]
            )

        def wait_out(l, b):
            pltpu.make_async_copy(
                outblk[b], out_hbm.at[l, :, pl.ds(b0, b_per_w)], osem[b]
            ).wait()

        lanes = lax.iota(jnp.int32, 16)
        skew = [jnp.bitwise_and(lanes + k, 15) for k in range(16)]

        def transpose_block(src, dst):
            def g_body(g, c):
                rows = lanes + g * 16
                for d0 in range(0, D, 16):
                    for k0 in range(0, 16, 8):
                        cvs = []
                        for k in range(k0, k0 + 8):
                            cols = skew[k] + d0
                            cvs.append(
                                (cols, plsc.load_gather(src, [rows, cols]))
                            )
                        for cols, v in cvs:
                            plsc.store_scatter(dst, [cols, rows], v)
                return c

            lax.fori_loop(0, 8, g_body, 0)

        start_gather(0, 0)
        start_gather(1, 1)
        start_gather(2, 2)

        def trip_body(p, carry):
            for b in range(3):
                l = p * 3 + b
                wait_gather(l, b)

                @pl.when(l >= 3)
                def _():
                    wait_out(l - 3, b)

                transpose_block(pairs[b], outblk[b])
                start_out(l, b)

                @pl.when(l + 3 < L)
                def _():
                    start_gather(l + 3, b)

            return carry

        lax.fori_loop(0, L // 3, trip_body, 0)

        for t, b in ((L - 2, (L - 2) % 3), (L - 1, (L - 1) % 3)):
            wait_gather(t, b)
            wait_out(t - 3, b)
            transpose_block(pairs[b], outblk[b])
            start_out(t, b)
        for t in range(L - 3, L):
            wait_out(t, t % 3)

    return sc_gather


def kernel(x, table):
    B, L = x.shape
    V, D = table.shape
    table_rm = _row_major_table(table.T)
    xt = x.T.astype(jnp.int32)
    out2 = _make_sc_gather(B, L, D, V)(xt, table_rm)
    return jnp.transpose(out2, (2, 0, 1))
